# baseline (speedup 1.0000x reference)
import functools

import jax
import jax.numpy as jnp
from jax import lax
from jax.experimental import pallas as pl
from jax.experimental.pallas import tpu as pltpu
from jax.experimental.pallas import tpu_sc as plsc

N = 16384
NUM_GROUPS = 4
LANES = 16
NUM_TILES = 16
CHUNK = N // NUM_TILES
VECS = CHUNK // LANES

_mesh = plsc.VectorSubcoreMesh(core_axis_name="c", subcore_axis_name="s")


@functools.partial(
    pl.kernel,
    mesh=_mesh,
    out_type=jax.ShapeDtypeStruct((LANES,), jnp.float32),
    compiler_params=pltpu.CompilerParams(needs_layout_passes=False),
    scratch_types=[
        pltpu.VMEM((CHUNK,), jnp.float32),
        pltpu.VMEM((CHUNK,), jnp.int32),
        pltpu.VMEM((LANES,), jnp.float32),
        pltpu.VMEM((LANES,), jnp.float32),
        pltpu.VMEM((NUM_TILES * 2 * LANES,), jnp.float32),
        pltpu.VMEM((LANES,), jnp.float32),
        pltpu.VMEM_SHARED((NUM_TILES * 2 * LANES,), jnp.float32),
    ],
)
def _fair_loss_sc(y_hbm, p_hbm, out_hbm, y_v, p_v, acc_s_v, acc_c_v,
                  all_v, out_v, shared):
    c = lax.axis_index("c")
    s = lax.axis_index("s")

    pltpu.sync_copy(y_hbm.at[pl.ds(s * CHUNK, CHUNK)], y_v)
    pltpu.sync_copy(p_hbm.at[pl.ds(s * CHUNK, CHUNK)], p_v)

    zeros = jnp.zeros((LANES,), jnp.float32)
    ones = jnp.ones((LANES,), jnp.float32)
    acc_s_v[...] = zeros
    acc_c_v[...] = zeros

    def body(i, carry):
        y = y_v[pl.ds(i * LANES, LANES)]
        p = p_v[pl.ds(i * LANES, LANES)]
        plsc.addupdate_scatter(acc_s_v, [p], y)
        plsc.addupdate_scatter(acc_c_v, [p], ones)
        return carry

    lax.fori_loop(0, VECS, body, 0)

    pltpu.sync_copy(acc_s_v, shared.at[pl.ds(s * 2 * LANES, LANES)])
    pltpu.sync_copy(acc_c_v, shared.at[pl.ds(s * 2 * LANES + LANES, LANES)])
    plsc.subcore_barrier()

    @pl.when(jnp.logical_and(s == 0, c == 0))
    def _():
        pltpu.sync_copy(shared, all_v)
        acc_s = zeros
        acc_c = zeros
        for t in range(NUM_TILES):
            acc_s = acc_s + all_v[pl.ds(t * 2 * LANES, LANES)]
            acc_c = acc_c + all_v[pl.ds(t * 2 * LANES + LANES, LANES)]
        means_v = acc_s / acc_c
        means = [means_v[g] for g in range(NUM_GROUPS)]
        mx = means[0]
        mn = means[0]
        for g in range(1, NUM_GROUPS):
            mx = jnp.maximum(mx, means[g])
            mn = jnp.minimum(mn, means[g])
        d = mx - mn
        loss = jnp.maximum(jnp.float32(0.0), d * d)
        out_v[...] = jnp.broadcast_to(loss, (LANES,))
        pltpu.sync_copy(out_v, out_hbm)


def kernel(y_label, y_pred, protected):
    del y_label
    out = _fair_loss_sc(y_pred.astype(jnp.float32),
                        protected.astype(jnp.int32))
    return out[0]


# speedup vs baseline: 1.1093x; 1.1093x over previous
import functools

import jax
import jax.numpy as jnp
from jax import lax
from jax.experimental import pallas as pl
from jax.experimental.pallas import tpu as pltpu
from jax.experimental.pallas import tpu_sc as plsc

N = 16384
NUM_GROUPS = 4
LANES = 16
NUM_TILES = 16
CHUNK = N // NUM_TILES
VECS = CHUNK // LANES

_mesh = plsc.VectorSubcoreMesh(
    core_axis_name="c", subcore_axis_name="s", num_cores=1)


@functools.partial(
    pl.kernel,
    mesh=_mesh,
    out_type=jax.ShapeDtypeStruct((LANES,), jnp.float32),
    compiler_params=pltpu.CompilerParams(needs_layout_passes=False),
    scratch_types=[
        pltpu.VMEM((CHUNK,), jnp.float32),
        pltpu.VMEM((CHUNK,), jnp.int32),
        pltpu.VMEM((LANES,), jnp.float32),
        pltpu.VMEM((LANES,), jnp.float32),
        pltpu.VMEM((2 * LANES,), jnp.float32),
        pltpu.VMEM((NUM_TILES * 2 * LANES,), jnp.float32),
        pltpu.VMEM((LANES,), jnp.float32),
        pltpu.VMEM_SHARED((NUM_TILES * 2 * LANES,), jnp.float32),
        pltpu.SemaphoreType.DMA,
        pltpu.SemaphoreType.DMA,
    ],
)
def _fair_loss_sc(y_hbm, p_hbm, out_hbm, y_v, p_v, acc_s_v, acc_c_v,
                  part_v, all_v, out_v, shared, sem0, sem1):
    s = lax.axis_index("s")

    cp_y = pltpu.async_copy(y_hbm.at[pl.ds(s * CHUNK, CHUNK)], y_v, sem0)
    cp_p = pltpu.async_copy(p_hbm.at[pl.ds(s * CHUNK, CHUNK)], p_v, sem1)

    zeros = jnp.zeros((LANES,), jnp.float32)
    ones = jnp.ones((LANES,), jnp.float32)
    acc_s_v[...] = zeros
    acc_c_v[...] = zeros
    cp_y.wait()
    cp_p.wait()

    def body(i, carry):
        y = y_v[pl.ds(i * LANES, LANES)]
        p = p_v[pl.ds(i * LANES, LANES)]
        plsc.addupdate_scatter(acc_s_v, [p], y)
        plsc.addupdate_scatter(acc_c_v, [p], ones)
        return carry

    lax.fori_loop(0, VECS, body, 0)

    part_v[pl.ds(0, LANES)] = acc_s_v[...]
    part_v[pl.ds(LANES, LANES)] = acc_c_v[...]
    pltpu.sync_copy(part_v, shared.at[pl.ds(s * 2 * LANES, 2 * LANES)])
    plsc.subcore_barrier()

    @pl.when(s == 0)
    def _():
        pltpu.sync_copy(shared, all_v)
        acc_s = zeros
        acc_c = zeros
        for t in range(NUM_TILES):
            acc_s = acc_s + all_v[pl.ds(t * 2 * LANES, LANES)]
            acc_c = acc_c + all_v[pl.ds(t * 2 * LANES + LANES, LANES)]

        means_v = acc_s / acc_c
        means = [means_v[g] for g in range(NUM_GROUPS)]
        mx = means[0]
        mn = means[0]
        for g in range(1, NUM_GROUPS):
            mx = jnp.maximum(mx, means[g])
            mn = jnp.minimum(mn, means[g])
        d = mx - mn
        loss = jnp.maximum(jnp.float32(0.0), d * d)
        out_v[...] = jnp.broadcast_to(loss, (LANES,))
        pltpu.sync_copy(out_v, out_hbm)


def kernel(y_label, y_pred, protected):
    del y_label
    out = _fair_loss_sc(y_pred.astype(jnp.float32),
                        protected.astype(jnp.int32))
    return out[0]


# speedup vs baseline: 1.1112x; 1.0016x over previous
import functools

import jax
import jax.numpy as jnp
from jax import lax
from jax.experimental import pallas as pl
from jax.experimental.pallas import tpu as pltpu
from jax.experimental.pallas import tpu_sc as plsc

N = 16384
NUM_GROUPS = 4
LANES = 16
NUM_TILES = 16
CHUNK = N // NUM_TILES
VECS = CHUNK // LANES

_mesh = plsc.VectorSubcoreMesh(
    core_axis_name="c", subcore_axis_name="s", num_cores=1)


@functools.partial(
    pl.kernel,
    mesh=_mesh,
    out_type=jax.ShapeDtypeStruct((LANES,), jnp.float32),
    compiler_params=pltpu.CompilerParams(needs_layout_passes=False),
    scratch_types=[
        pltpu.VMEM((CHUNK,), jnp.float32),
        pltpu.VMEM((CHUNK,), jnp.int32),
        pltpu.VMEM((2 * LANES,), jnp.float32),
        pltpu.VMEM_SHARED((NUM_TILES * 2 * LANES,), jnp.float32),
        pltpu.SemaphoreType.DMA,
    ],
)
def _fair_loss_sc(y_hbm, p_hbm, out_hbm, y_v, p_v, acc_v, shared, sem):
    s = lax.axis_index("s")

    cp_y = pltpu.async_copy(y_hbm.at[pl.ds(s * CHUNK, CHUNK)], y_v, sem)
    cp_p = pltpu.async_copy(p_hbm.at[pl.ds(s * CHUNK, CHUNK)], p_v, sem)

    zeros = jnp.zeros((LANES,), jnp.float32)
    ones = jnp.ones((LANES,), jnp.float32)
    sixteen = jnp.full((LANES,), LANES, jnp.int32)
    acc_v[pl.ds(0, LANES)] = zeros
    acc_v[pl.ds(LANES, LANES)] = zeros
    cp_y.wait()
    cp_p.wait()

    def body(i, carry):
        y = y_v[pl.ds(i * LANES, LANES)]
        p = p_v[pl.ds(i * LANES, LANES)]
        plsc.addupdate_scatter(acc_v, [p], y)
        plsc.addupdate_scatter(acc_v, [p + sixteen], ones)
        return carry

    lax.fori_loop(0, VECS, body, 0)

    pltpu.sync_copy(acc_v, shared.at[pl.ds(s * 2 * LANES, 2 * LANES)])
    plsc.subcore_barrier()

    @pl.when(s == 0)
    def _():
        pltpu.sync_copy(shared, y_v.at[pl.ds(0, NUM_TILES * 2 * LANES)])
        acc_s = zeros
        acc_c = zeros
        for t in range(NUM_TILES):
            acc_s = acc_s + y_v[pl.ds(t * 2 * LANES, LANES)]
            acc_c = acc_c + y_v[pl.ds(t * 2 * LANES + LANES, LANES)]

        means_v = acc_s / acc_c
        means = [means_v[g] for g in range(NUM_GROUPS)]
        mx = means[0]
        mn = means[0]
        for g in range(1, NUM_GROUPS):
            mx = jnp.maximum(mx, means[g])
            mn = jnp.minimum(mn, means[g])
        d = mx - mn
        loss = jnp.maximum(jnp.float32(0.0), d * d)
        acc_v[pl.ds(0, LANES)] = jnp.broadcast_to(loss, (LANES,))
        pltpu.sync_copy(acc_v.at[pl.ds(0, LANES)], out_hbm)


def kernel(y_label, y_pred, protected):
    del y_label
    out = _fair_loss_sc(y_pred.astype(jnp.float32),
                        protected.astype(jnp.int32))
    return out[0]
